# baseline (speedup 1.0000x reference)
import functools

import jax
import jax.numpy as jnp
from jax import lax
from jax.experimental import pallas as pl
from jax.experimental.pallas import tpu as pltpu
from jax.experimental.pallas import tpu_sc as plsc

_CENTROIDS = (
    (36.68147669256268, -82.8910274009993),
    (23.22243322909555, 78.23027450833709),
    (50.04997682638993, 0.22379313938744885),
    (37.9309447099281, -117.00741350764692),
    (-32.795864819917725, 148.7159172660312),
    (-18.570548393114084, -54.280255665692565),
    (13.921140442819565, 116.38740315555172),
    (29.78951080730802, 40.279515865947936),
)
_AGE_BOUNDS = (18.0, 25.0, 30.0, 35.0, 40.0, 45.0, 50.0, 55.0, 60.0, 65.0)

_NC = 2
_NS = 16
_NW = _NC * _NS
_LANES = 16
_D = 32

_BASE_G, _BASE_L, _BASE_C, _BASE_N, _BASE_A, _BASE_LL = 0, 24, 240, 656, 704, 728
_FT_LEN = 752


def _sc_tower(g, l, c, n, a, lat_col, lon_col, tstack):
  B = g.shape[0]
  bpw = B // _NW
  groups = bpw // _LANES

  mesh = plsc.VectorSubcoreMesh(
      core_axis_name="c", subcore_axis_name="s",
      num_cores=_NC, num_subcores=_NS)

  @functools.partial(
      pl.kernel,
      mesh=mesh,
      compiler_params=pltpu.CompilerParams(needs_layout_passes=False),
      out_type=jax.ShapeDtypeStruct((B, _D), jnp.float32),
      scratch_types=[
          pltpu.VMEM((bpw,), jnp.int32),
          pltpu.VMEM((bpw,), jnp.int32),
          pltpu.VMEM((bpw,), jnp.int32),
          pltpu.VMEM((bpw,), jnp.int32),
          pltpu.VMEM((bpw,), jnp.int32),
          pltpu.VMEM((bpw,), jnp.float32),
          pltpu.VMEM((bpw,), jnp.float32),
          pltpu.VMEM((6, 48, 128), jnp.float32),
          pltpu.VMEM((bpw * 33,), jnp.float32),
          pltpu.VMEM((bpw, _D), jnp.float32),
          pltpu.SemaphoreType.DMA,
      ],
  )
  def tower(g_h, l_h, c_h, n_h, a_h, lat_h, lon_h, ts_h,
            out_h, g_v, l_v, c_v, n_v, a_v, lat_v, lon_v,
            ft_v, s33_v, stg_v, sem):
    wid = lax.axis_index("s") * _NC + lax.axis_index("c")
    ub = wid * bpw

    copies = [
        pltpu.async_copy(g_h.at[pl.ds(ub, bpw)], g_v, sem),
        pltpu.async_copy(l_h.at[pl.ds(ub, bpw)], l_v, sem),
        pltpu.async_copy(c_h.at[pl.ds(ub, bpw)], c_v, sem),
        pltpu.async_copy(n_h.at[pl.ds(ub, bpw)], n_v, sem),
        pltpu.async_copy(a_h.at[pl.ds(ub, bpw)], a_v, sem),
        pltpu.async_copy(lat_h.at[pl.ds(ub, bpw)], lat_v, sem),
        pltpu.async_copy(lon_h.at[pl.ds(ub, bpw)], lon_v, sem),
        pltpu.async_copy(ts_h, ft_v, sem),
    ]
    for cp in copies:
      cp.wait()

    lanes = lax.broadcasted_iota(jnp.int32, (_LANES,), 0)

    @plsc.parallel_loop(0, groups, step=1, unroll=2)
    def group(i):
      u0 = i * _LANES
      uvec = u0 + lanes
      gi = g_v[pl.ds(u0, _LANES)]
      li = l_v[pl.ds(u0, _LANES)]
      ci = c_v[pl.ds(u0, _LANES)]
      ni = n_v[pl.ds(u0, _LANES)]
      ai = a_v[pl.ds(u0, _LANES)]
      lat = lat_v[pl.ds(u0, _LANES)]
      lon = lon_v[pl.ds(u0, _LANES)]

      af = ai.astype(jnp.float32)
      aidx = jnp.zeros((_LANES,), jnp.int32)
      for b in _AGE_BOUNDS:
        aidx = aidx + (af >= b).astype(jnp.int32)

      dlat = lat - _CENTROIDS[0][0]
      dlon = lon - _CENTROIDS[0][1]
      best_d = dlat * dlat + dlon * dlon
      best_k = jnp.zeros((_LANES,), jnp.int32)
      for k in range(1, 8):
        dlat = lat - _CENTROIDS[k][0]
        dlon = lon - _CENTROIDS[k][1]
        d = dlat * dlat + dlon * dlon
        m = d < best_d
        best_k = jnp.where(m, k, best_k)
        best_d = jnp.where(m, d, best_d)

      feats = (
          (0, gi, 4),
          (1, li, 10),
          (2, ci, 10),
          (3, ni, 4),
          (4, aidx, 2),
          (5, best_k + 2, 2),
      )
      rb33 = uvec * 33
      zeros16 = lanes * 0
      col = 0
      for t, rows, width in feats:
        for j in range(width):
          val = plsc.load_gather(ft_v, [zeros16 + t, rows, zeros16 + j])
          plsc.store_scatter(s33_v, [rb33 + col], val)
          col += 1

    @plsc.parallel_loop(0, bpw, step=1, unroll=4)
    def compact(u):
      stg_v[u, pl.ds(0, _LANES)] = s33_v[pl.ds(u * 33, _LANES)]
      stg_v[u, pl.ds(_LANES, _LANES)] = s33_v[pl.ds(u * 33 + _LANES, _LANES)]

    pltpu.sync_copy(stg_v, out_h.at[pl.ds(ub, bpw)])

  return tower(g, l, c, n, a, lat_col, lon_col, tstack)


def kernel(viewer_gender, viewer_lang, viewer_country, viewer_network,
           viewer_age, viewer_lat_long, gender_table, lang_table,
           country_table, network_table, age_table, latlong_table):
  tstack = jnp.stack([
      jnp.pad(gender_table, ((0, 43), (0, 124))),
      jnp.pad(lang_table, ((0, 27), (0, 118))),
      jnp.pad(country_table, ((0, 7), (0, 118))),
      jnp.pad(network_table, ((0, 37), (0, 124))),
      jnp.pad(age_table, ((0, 37), (0, 126))),
      jnp.pad(latlong_table, ((0, 38), (0, 126))),
  ])
  return _sc_tower(
      viewer_gender, viewer_lang, viewer_country, viewer_network,
      viewer_age, viewer_lat_long[:, 0], viewer_lat_long[:, 1], tstack)


# speedup vs baseline: 1.0265x; 1.0265x over previous
import functools

import jax
import jax.numpy as jnp
from jax import lax
from jax.experimental import pallas as pl
from jax.experimental.pallas import tpu as pltpu
from jax.experimental.pallas import tpu_sc as plsc

_CENTROIDS = (
    (36.68147669256268, -82.8910274009993),
    (23.22243322909555, 78.23027450833709),
    (50.04997682638993, 0.22379313938744885),
    (37.9309447099281, -117.00741350764692),
    (-32.795864819917725, 148.7159172660312),
    (-18.570548393114084, -54.280255665692565),
    (13.921140442819565, 116.38740315555172),
    (29.78951080730802, 40.279515865947936),
)
_AGE_BOUNDS = (18.0, 25.0, 30.0, 35.0, 40.0, 45.0, 50.0, 55.0, 60.0, 65.0)

_NC = 2
_NS = 16
_NW = _NC * _NS
_LANES = 16
_D = 32

_BASE_G, _BASE_L, _BASE_C, _BASE_N, _BASE_A, _BASE_LL = 0, 24, 240, 656, 704, 728
_FT_LEN = 752


def _sc_tower(g, l, c, n, a, lat_col, lon_col, tstack):
  B = g.shape[0]
  bpw = B // _NW
  groups = bpw // _LANES

  mesh = plsc.VectorSubcoreMesh(
      core_axis_name="c", subcore_axis_name="s",
      num_cores=_NC, num_subcores=_NS)

  @functools.partial(
      pl.kernel,
      mesh=mesh,
      compiler_params=pltpu.CompilerParams(needs_layout_passes=False),
      out_type=jax.ShapeDtypeStruct((B, _D), jnp.float32),
      scratch_types=[
          pltpu.VMEM((bpw,), jnp.int32),
          pltpu.VMEM((bpw,), jnp.int32),
          pltpu.VMEM((bpw,), jnp.int32),
          pltpu.VMEM((bpw,), jnp.int32),
          pltpu.VMEM((bpw,), jnp.int32),
          pltpu.VMEM((bpw,), jnp.float32),
          pltpu.VMEM((bpw,), jnp.float32),
          pltpu.VMEM((48, _D), jnp.float32),
          pltpu.VMEM((bpw * 33,), jnp.float32),
          pltpu.VMEM((bpw, _D), jnp.float32),
          pltpu.SemaphoreType.DMA,
      ],
  )
  def tower(g_h, l_h, c_h, n_h, a_h, lat_h, lon_h, ts_h,
            out_h, g_v, l_v, c_v, n_v, a_v, lat_v, lon_v,
            ft_v, s33_v, stg_v, sem):
    wid = lax.axis_index("s") * _NC + lax.axis_index("c")
    ub = wid * bpw

    copies = [
        pltpu.async_copy(g_h.at[pl.ds(ub, bpw)], g_v, sem),
        pltpu.async_copy(l_h.at[pl.ds(ub, bpw)], l_v, sem),
        pltpu.async_copy(c_h.at[pl.ds(ub, bpw)], c_v, sem),
        pltpu.async_copy(n_h.at[pl.ds(ub, bpw)], n_v, sem),
        pltpu.async_copy(a_h.at[pl.ds(ub, bpw)], a_v, sem),
        pltpu.async_copy(lat_h.at[pl.ds(ub, bpw)], lat_v, sem),
        pltpu.async_copy(lon_h.at[pl.ds(ub, bpw)], lon_v, sem),
        pltpu.async_copy(ts_h, ft_v, sem),
    ]
    for cp in copies:
      cp.wait()

    lanes = lax.broadcasted_iota(jnp.int32, (_LANES,), 0)

    @plsc.parallel_loop(0, groups, step=1, unroll=2)
    def group(i):
      u0 = i * _LANES
      uvec = u0 + lanes
      gi = g_v[pl.ds(u0, _LANES)]
      li = l_v[pl.ds(u0, _LANES)]
      ci = c_v[pl.ds(u0, _LANES)]
      ni = n_v[pl.ds(u0, _LANES)]
      ai = a_v[pl.ds(u0, _LANES)]
      lat = lat_v[pl.ds(u0, _LANES)]
      lon = lon_v[pl.ds(u0, _LANES)]

      af = ai.astype(jnp.float32)
      aidx = jnp.zeros((_LANES,), jnp.int32)
      for b in _AGE_BOUNDS:
        aidx = aidx + (af >= b).astype(jnp.int32)

      dlat = lat - _CENTROIDS[0][0]
      dlon = lon - _CENTROIDS[0][1]
      best_d = dlat * dlat + dlon * dlon
      best_k = jnp.zeros((_LANES,), jnp.int32)
      for k in range(1, 8):
        dlat = lat - _CENTROIDS[k][0]
        dlon = lon - _CENTROIDS[k][1]
        d = dlat * dlat + dlon * dlon
        m = d < best_d
        best_k = jnp.where(m, k, best_k)
        best_d = jnp.where(m, d, best_d)

      feats = (
          (gi, 4),
          (li, 10),
          (ci, 10),
          (ni, 4),
          (aidx, 2),
          (best_k + 2, 2),
      )
      rb33 = uvec * 33
      zeros16 = lanes * 0
      col = 0
      for rows, width in feats:
        for j in range(width):
          val = plsc.load_gather(ft_v, [rows, zeros16 + col])
          plsc.store_scatter(s33_v, [rb33 + col], val)
          col += 1

    @plsc.parallel_loop(0, bpw, step=1, unroll=4)
    def compact(u):
      stg_v[u, pl.ds(0, _LANES)] = s33_v[pl.ds(u * 33, _LANES)]
      stg_v[u, pl.ds(_LANES, _LANES)] = s33_v[pl.ds(u * 33 + _LANES, _LANES)]

    pltpu.sync_copy(stg_v, out_h.at[pl.ds(ub, bpw)])

  return tower(g, l, c, n, a, lat_col, lon_col, tstack)


def kernel(viewer_gender, viewer_lang, viewer_country, viewer_network,
           viewer_age, viewer_lat_long, gender_table, lang_table,
           country_table, network_table, age_table, latlong_table):
  tstack = jnp.concatenate([
      jnp.pad(gender_table, ((0, 43), (0, 0))),
      jnp.pad(lang_table, ((0, 27), (0, 0))),
      jnp.pad(country_table, ((0, 7), (0, 0))),
      jnp.pad(network_table, ((0, 37), (0, 0))),
      jnp.pad(age_table, ((0, 37), (0, 0))),
      jnp.pad(latlong_table, ((0, 38), (0, 0))),
  ], axis=1)
  return _sc_tower(
      viewer_gender, viewer_lang, viewer_country, viewer_network,
      viewer_age, viewer_lat_long[:, 0], viewer_lat_long[:, 1], tstack)


# speedup vs baseline: 1.1598x; 1.1299x over previous
import functools

import jax
import jax.numpy as jnp
from jax import lax
from jax.experimental import pallas as pl
from jax.experimental.pallas import tpu as pltpu
from jax.experimental.pallas import tpu_sc as plsc

_CENTROIDS = (
    (36.68147669256268, -82.8910274009993),
    (23.22243322909555, 78.23027450833709),
    (50.04997682638993, 0.22379313938744885),
    (37.9309447099281, -117.00741350764692),
    (-32.795864819917725, 148.7159172660312),
    (-18.570548393114084, -54.280255665692565),
    (13.921140442819565, 116.38740315555172),
    (29.78951080730802, 40.279515865947936),
)
_AGE_BOUNDS = (18.0, 25.0, 30.0, 35.0, 40.0, 45.0, 50.0, 55.0, 60.0, 65.0)

_NC = 2
_NS = 16
_NW = _NC * _NS
_LANES = 16
_D = 32

_BASE_G, _BASE_L, _BASE_C, _BASE_N, _BASE_A, _BASE_LL = 0, 24, 240, 656, 704, 728
_FT_LEN = 752


def _sc_tower(g, l, c, n, a, lat_col, lon_col, gt, lt, ct, nt, at_, llt):
  B = g.shape[0]
  bpw = B // _NW
  groups = bpw // _LANES

  mesh = plsc.VectorSubcoreMesh(
      core_axis_name="c", subcore_axis_name="s",
      num_cores=_NC, num_subcores=_NS)

  @functools.partial(
      pl.kernel,
      mesh=mesh,
      compiler_params=pltpu.CompilerParams(needs_layout_passes=False),
      out_type=jax.ShapeDtypeStruct((B, _D), jnp.float32),
      scratch_types=[
          pltpu.VMEM((bpw,), jnp.int32),
          pltpu.VMEM((bpw,), jnp.int32),
          pltpu.VMEM((bpw,), jnp.int32),
          pltpu.VMEM((bpw,), jnp.int32),
          pltpu.VMEM((bpw,), jnp.int32),
          pltpu.VMEM((bpw,), jnp.float32),
          pltpu.VMEM((bpw,), jnp.float32),
          pltpu.VMEM((_FT_LEN,), jnp.float32),
          pltpu.VMEM((bpw * 33,), jnp.float32),
          pltpu.VMEM((bpw, _D), jnp.float32),
          pltpu.SemaphoreType.DMA,
      ],
  )
  def tower(g_h, l_h, c_h, n_h, a_h, lat_h, lon_h, gt_h, lt_h, ct_h, nt_h,
            at_h, llt_h, out_h, g_v, l_v, c_v, n_v, a_v, lat_v, lon_v,
            ft_v, s33_v, stg_v, sem):
    wid = lax.axis_index("s") * _NC + lax.axis_index("c")
    ub = wid * bpw

    copies = [
        pltpu.async_copy(g_h.at[pl.ds(ub, bpw)], g_v, sem),
        pltpu.async_copy(l_h.at[pl.ds(ub, bpw)], l_v, sem),
        pltpu.async_copy(c_h.at[pl.ds(ub, bpw)], c_v, sem),
        pltpu.async_copy(n_h.at[pl.ds(ub, bpw)], n_v, sem),
        pltpu.async_copy(a_h.at[pl.ds(ub, bpw)], a_v, sem),
        pltpu.async_copy(lat_h.at[pl.ds(ub, bpw)], lat_v, sem),
        pltpu.async_copy(lon_h.at[pl.ds(ub, bpw)], lon_v, sem),
        pltpu.async_copy(gt_h, ft_v.at[pl.ds(_BASE_G, 20)], sem),
        pltpu.async_copy(lt_h, ft_v.at[pl.ds(_BASE_L, 210)], sem),
        pltpu.async_copy(ct_h, ft_v.at[pl.ds(_BASE_C, 410)], sem),
        pltpu.async_copy(nt_h, ft_v.at[pl.ds(_BASE_N, 44)], sem),
        pltpu.async_copy(at_h, ft_v.at[pl.ds(_BASE_A, 22)], sem),
        pltpu.async_copy(llt_h, ft_v.at[pl.ds(_BASE_LL, 20)], sem),
    ]
    for cp in copies:
      cp.wait()

    lanes = lax.broadcasted_iota(jnp.int32, (_LANES,), 0)

    @plsc.parallel_loop(0, groups, step=1, unroll=2)
    def group(i):
      u0 = i * _LANES
      uvec = u0 + lanes
      gi = g_v[pl.ds(u0, _LANES)]
      li = l_v[pl.ds(u0, _LANES)]
      ci = c_v[pl.ds(u0, _LANES)]
      ni = n_v[pl.ds(u0, _LANES)]
      ai = a_v[pl.ds(u0, _LANES)]
      lat = lat_v[pl.ds(u0, _LANES)]
      lon = lon_v[pl.ds(u0, _LANES)]

      af = ai.astype(jnp.float32)
      aidx = jnp.zeros((_LANES,), jnp.int32)
      for b in _AGE_BOUNDS:
        aidx = aidx + (af >= b).astype(jnp.int32)

      dlat = lat - _CENTROIDS[0][0]
      dlon = lon - _CENTROIDS[0][1]
      best_d = dlat * dlat + dlon * dlon
      best_k = jnp.zeros((_LANES,), jnp.int32)
      for k in range(1, 8):
        dlat = lat - _CENTROIDS[k][0]
        dlon = lon - _CENTROIDS[k][1]
        d = dlat * dlat + dlon * dlon
        m = d < best_d
        best_k = jnp.where(m, k, best_k)
        best_d = jnp.where(m, d, best_d)

      row_offs = (
          (gi * 4 + _BASE_G, 4),
          (li * 10 + _BASE_L, 10),
          (ci * 10 + _BASE_C, 10),
          (ni * 4 + _BASE_N, 4),
          (aidx * 2 + _BASE_A, 2),
          ((best_k + 2) * 2 + _BASE_LL, 2),
      )
      rb33 = uvec * 33
      col = 0
      for base, width in row_offs:
        for j in range(width):
          val = plsc.load_gather(ft_v, [base + j])
          plsc.store_scatter(s33_v, [rb33 + col], val)
          col += 1

    @plsc.parallel_loop(0, bpw, step=1, unroll=4)
    def compact(u):
      stg_v[u, pl.ds(0, _LANES)] = s33_v[pl.ds(u * 33, _LANES)]
      stg_v[u, pl.ds(_LANES, _LANES)] = s33_v[pl.ds(u * 33 + _LANES, _LANES)]

    pltpu.sync_copy(stg_v, out_h.at[pl.ds(ub, bpw)])

  return tower(g, l, c, n, a, lat_col, lon_col, gt, lt, ct, nt, at_, llt)


def kernel(viewer_gender, viewer_lang, viewer_country, viewer_network,
           viewer_age, viewer_lat_long, gender_table, lang_table,
           country_table, network_table, age_table, latlong_table):
  return _sc_tower(
      viewer_gender, viewer_lang, viewer_country, viewer_network,
      viewer_age, viewer_lat_long[:, 0], viewer_lat_long[:, 1],
      gender_table.reshape(-1), lang_table.reshape(-1),
      country_table.reshape(-1), network_table.reshape(-1),
      age_table.reshape(-1), latlong_table.reshape(-1))
